# Initial kernel scaffold; baseline (speedup 1.0000x reference)
import functools

import jax
import jax.numpy as jnp
from jax import lax
from jax.experimental import pallas as pl
from jax.experimental.pallas import tpu as pltpu
from jax.experimental.pallas import tpu_sc as plsc

NC = 2
NS = 16


def _sc_aggregate(x, src3, dst3, zeros_blk):
    n, d = x.shape
    _, n_iter, batch = src3.shape
    rows_per_tile = n // NS

    mesh = plsc.VectorSubcoreMesh(
        core_axis_name="c", subcore_axis_name="s", num_cores=NC, num_subcores=NS
    )

    @functools.partial(
        pl.kernel,
        out_type=jax.ShapeDtypeStruct((NC, n, d), jnp.float32),
        mesh=mesh,
        scratch_types=[
            pltpu.VMEM((n_iter, batch), jnp.int32),
            pltpu.VMEM((n_iter, batch), jnp.int32),
            pltpu.VMEM((batch, d), jnp.float32),
            pltpu.VMEM_SHARED((n, d), jnp.float32),
            pltpu.SemaphoreType.DMA,
        ],
    )
    def agg_kernel(x_hbm, src_hbm, dst_hbm, zeros_hbm, out_hbm,
                   src_v, dst_v, rows_v, acc_sh, sem):
        c = lax.axis_index("c")
        s = lax.axis_index("s")
        wid = c * NS + s
        row0 = s * rows_per_tile

        pltpu.sync_copy(zeros_hbm, acc_sh.at[pl.ds(row0, rows_per_tile)])
        pltpu.sync_copy(src_hbm.at[wid], src_v)
        pltpu.sync_copy(dst_hbm.at[wid], dst_v)

        plsc.subcore_barrier()

        @pl.loop(0, n_iter)
        def _edges(j):
            pltpu.async_copy(x_hbm.at[src_v.at[j]], rows_v, sem).wait()
            pltpu.sync_copy(rows_v, acc_sh.at[dst_v.at[j]], add=True)

        plsc.subcore_barrier()
        pltpu.sync_copy(acc_sh.at[pl.ds(row0, rows_per_tile)],
                        out_hbm.at[c, pl.ds(row0, rows_per_tile)])

    return agg_kernel(x, src3, dst3, zeros_blk)


def _mlp_body(x_ref, a0_ref, a1_ref, wa_ref, ba_ref, wb_ref, bb_ref, o_ref):
    h = x_ref[...] + a0_ref[...] + a1_ref[...]
    h = jnp.maximum(
        jnp.dot(h, wa_ref[...], preferred_element_type=jnp.float32)
        + ba_ref[...], 0.0)
    h = jnp.maximum(
        jnp.dot(h, wb_ref[...], preferred_element_type=jnp.float32)
        + bb_ref[...], 0.0)
    o_ref[...] = h


def _mlp2_body(x_ref, a0_ref, a1_ref, wa_ref, ba_ref, wb_ref, bb_ref,
               wfc_ref, bfc_ref, o_ref):
    h = x_ref[...] + a0_ref[...] + a1_ref[...]
    h = jnp.maximum(
        jnp.dot(h, wa_ref[...], preferred_element_type=jnp.float32)
        + ba_ref[...], 0.0)
    h = jnp.maximum(
        jnp.dot(h, wb_ref[...], preferred_element_type=jnp.float32)
        + bb_ref[...], 0.0)
    logits = (jnp.dot(h, wfc_ref[...], preferred_element_type=jnp.float32)
              + bfc_ref[...])
    m = jnp.max(logits, axis=1, keepdims=True)
    lse = jnp.log(jnp.sum(jnp.exp(logits - m), axis=1, keepdims=True)) + m
    o_ref[...] = logits - lse


def _specs(bn, d, n_mats):
    row = pl.BlockSpec((bn, d), lambda i: (i, 0))
    mat = pl.BlockSpec((d, d), lambda i: (0, 0))
    vec = pl.BlockSpec((1, d), lambda i: (0, 0))
    return [row, row, row] + [mat, vec] * n_mats


def _mlp(x, a0, a1, wa, ba, wb, bb, bn):
    n, d = x.shape
    return pl.pallas_call(
        _mlp_body,
        grid=(n // bn,),
        in_specs=_specs(bn, d, 2),
        out_specs=pl.BlockSpec((bn, d), lambda i: (i, 0)),
        out_shape=jax.ShapeDtypeStruct((n, d), jnp.float32),
    )(x, a0, a1, wa, ba.reshape(1, d), wb, bb.reshape(1, d))


def _mlp2_fc_logsoftmax(x, a0, a1, wa, ba, wb, bb, wfc_p, bfc_p, bn):
    n, d = x.shape
    return pl.pallas_call(
        _mlp2_body,
        grid=(n // bn,),
        in_specs=_specs(bn, d, 3),
        out_specs=pl.BlockSpec((bn, d), lambda i: (i, 0)),
        out_shape=jax.ShapeDtypeStruct((n, d), jnp.float32),
    )(x, a0, a1, wa, ba.reshape(1, d), wb, bb.reshape(1, d),
      wfc_p, bfc_p.reshape(1, d))


def kernel(x, edge_index, W1a, b1a, W1b, b1b, W2a, b2a, W2b, b2b, Wfc, bfc):
    n, d = x.shape
    e = edge_index.shape[1]
    c = Wfc.shape[1]

    nt = NC * NS
    e_per_w = e // nt
    batch = 125
    n_iter = e_per_w // batch
    src3 = edge_index[0].reshape(nt, n_iter, batch)
    dst3 = edge_index[1].reshape(nt, n_iter, batch)
    zeros_blk = jnp.zeros((n // NS, d), jnp.float32)

    wfc_p = jnp.zeros((d, d), jnp.float32).at[:, :c].set(Wfc)
    bfc_p = jnp.full((d,), -1e30, jnp.float32).at[:c].set(bfc)

    bn = 1250
    agg1 = _sc_aggregate(x, src3, dst3, zeros_blk)
    h1 = _mlp(x, agg1[0], agg1[1], W1a, b1a, W1b, b1b, bn)
    agg2 = _sc_aggregate(h1, src3, dst3, zeros_blk)
    out_p = _mlp2_fc_logsoftmax(h1, agg2[0], agg2[1], W2a, b2a, W2b, b2b,
                                wfc_p, bfc_p, bn)
    return out_p[:, :c]


# speedup vs baseline: 7.1966x; 7.1966x over previous
import functools

import jax
import jax.numpy as jnp
from jax import lax
from jax.experimental import pallas as pl
from jax.experimental.pallas import tpu as pltpu
from jax.experimental.pallas import tpu_sc as plsc

NC = 2
NS = 16


def _sc_aggregate(x, src3, dst3, zeros_blk):
    n, d = x.shape
    _, n_iter, batch = src3.shape
    rows_per_tile = zeros_blk.shape[0]
    n_pad = rows_per_tile * NS

    mesh = plsc.VectorSubcoreMesh(
        core_axis_name="c", subcore_axis_name="s", num_cores=NC, num_subcores=NS
    )

    @functools.partial(
        pl.kernel,
        out_type=jax.ShapeDtypeStruct((NC, n_pad, d), jnp.float32),
        mesh=mesh,
        scratch_types=[
            pltpu.VMEM((n_iter, batch), jnp.int32),
            pltpu.VMEM((n_iter, batch), jnp.int32),
            pltpu.VMEM((batch, d), jnp.float32),
            pltpu.VMEM_SHARED((n_pad, d), jnp.float32),
            pltpu.SemaphoreType.DMA,
        ],
    )
    def agg_kernel(x_hbm, src_hbm, dst_hbm, zeros_hbm, out_hbm,
                   src_v, dst_v, rows_v, acc_sh, sem):
        c = lax.axis_index("c")
        s = lax.axis_index("s")
        wid = c * NS + s
        row0 = s * rows_per_tile

        pltpu.sync_copy(zeros_hbm, acc_sh.at[pl.ds(row0, rows_per_tile)])
        pltpu.sync_copy(src_hbm.at[wid], src_v)
        pltpu.sync_copy(dst_hbm.at[wid], dst_v)

        plsc.subcore_barrier()

        @pl.loop(0, n_iter)
        def _edges(j):
            pltpu.async_copy(x_hbm.at[src_v.at[j]], rows_v, sem).wait()
            pltpu.sync_copy(rows_v, acc_sh.at[dst_v.at[j]], add=True)

        plsc.subcore_barrier()
        pltpu.sync_copy(acc_sh.at[pl.ds(row0, rows_per_tile)],
                        out_hbm.at[c, pl.ds(row0, rows_per_tile)])

    return agg_kernel(x, src3, dst3, zeros_blk)


def _mlp_body(x_ref, a0_ref, a1_ref, wa_ref, ba_ref, wb_ref, bb_ref, o_ref):
    h = x_ref[...] + a0_ref[...] + a1_ref[...]
    h = jnp.maximum(
        jnp.dot(h, wa_ref[...], preferred_element_type=jnp.float32)
        + ba_ref[...], 0.0)
    h = jnp.maximum(
        jnp.dot(h, wb_ref[...], preferred_element_type=jnp.float32)
        + bb_ref[...], 0.0)
    o_ref[...] = h


def _mlp2_body(x_ref, a0_ref, a1_ref, wa_ref, ba_ref, wb_ref, bb_ref,
               wfc_ref, bfc_ref, o_ref):
    h = x_ref[...] + a0_ref[...] + a1_ref[...]
    h = jnp.maximum(
        jnp.dot(h, wa_ref[...], preferred_element_type=jnp.float32)
        + ba_ref[...], 0.0)
    h = jnp.maximum(
        jnp.dot(h, wb_ref[...], preferred_element_type=jnp.float32)
        + bb_ref[...], 0.0)
    logits = (jnp.dot(h, wfc_ref[...], preferred_element_type=jnp.float32)
              + bfc_ref[...])
    m = jnp.max(logits, axis=1, keepdims=True)
    lse = jnp.log(jnp.sum(jnp.exp(logits - m), axis=1, keepdims=True)) + m
    o_ref[...] = logits - lse


def _specs(bn, d, n_mats):
    row = pl.BlockSpec((bn, d), lambda i: (i, 0))
    mat = pl.BlockSpec((d, d), lambda i: (0, 0))
    vec = pl.BlockSpec((1, d), lambda i: (0, 0))
    return [row, row, row] + [mat, vec] * n_mats


def _mlp(x, a0, a1, wa, ba, wb, bb, bn):
    n, d = x.shape
    return pl.pallas_call(
        _mlp_body,
        grid=(n // bn,),
        in_specs=_specs(bn, d, 2),
        out_specs=pl.BlockSpec((bn, d), lambda i: (i, 0)),
        out_shape=jax.ShapeDtypeStruct((n, d), jnp.float32),
    )(x, a0, a1, wa, ba.reshape(1, d), wb, bb.reshape(1, d))


def _mlp2_fc_logsoftmax(x, a0, a1, wa, ba, wb, bb, wfc_p, bfc_p, bn):
    n, d = x.shape
    return pl.pallas_call(
        _mlp2_body,
        grid=(n // bn,),
        in_specs=_specs(bn, d, 3),
        out_specs=pl.BlockSpec((bn, d), lambda i: (i, 0)),
        out_shape=jax.ShapeDtypeStruct((n, d), jnp.float32),
    )(x, a0, a1, wa, ba.reshape(1, d), wb, bb.reshape(1, d),
      wfc_p, bfc_p.reshape(1, d))


def kernel(x, edge_index, W1a, b1a, W1b, b1b, W2a, b2a, W2b, b2b, Wfc, bfc):
    n, d = x.shape
    e = edge_index.shape[1]
    c = Wfc.shape[1]

    nt = NC * NS
    e_per_w = e // nt
    batch = 125
    n_iter = e_per_w // batch
    src3 = edge_index[0].reshape(nt, n_iter, batch)
    dst3 = edge_index[1].reshape(nt, n_iter, batch)
    n_pad = -(-n // (NS * 8)) * (NS * 8)
    zeros_blk = jnp.zeros((n_pad // NS, d), jnp.float32)

    wfc_p = jnp.zeros((d, d), jnp.float32).at[:, :c].set(Wfc)
    bfc_p = jnp.full((d,), -1e30, jnp.float32).at[:c].set(bfc)

    bn = 1000
    agg1 = _sc_aggregate(x, src3, dst3, zeros_blk)
    h1 = _mlp(x, agg1[0], agg1[1], W1a, b1a, W1b, b1b, bn)
    agg2 = _sc_aggregate(h1, src3, dst3, zeros_blk)
    out_p = _mlp2_fc_logsoftmax(h1, agg2[0], agg2[1], W2a, b2a, W2b, b2b,
                                wfc_p, bfc_p, bn)
    return out_p[:, :c]


# speedup vs baseline: 9.8469x; 1.3683x over previous
import functools

import jax
import jax.numpy as jnp
from jax import lax
from jax.experimental import pallas as pl
from jax.experimental.pallas import tpu as pltpu
from jax.experimental.pallas import tpu_sc as plsc

NC = 2
NS = 16


def _sc_aggregate(x, src2, dst3, zeros_blk):
    n, d = x.shape
    _, n_iter, batch = dst3.shape
    rows_per_tile = zeros_blk.shape[0]
    n_pad = rows_per_tile * NS

    mesh = plsc.VectorSubcoreMesh(
        core_axis_name="c", subcore_axis_name="s", num_cores=NC, num_subcores=NS
    )

    @functools.partial(
        pl.kernel,
        out_type=jax.ShapeDtypeStruct((NC, n_pad, d), jnp.float32),
        mesh=mesh,
        scratch_types=[
            pltpu.VMEM((n_iter * batch,), jnp.int32),
            pltpu.VMEM((n_iter, batch), jnp.int32),
            pltpu.VMEM((batch, d), jnp.float32),
            pltpu.VMEM((batch, d), jnp.float32),
            pltpu.VMEM_SHARED((n_pad, d), jnp.float32),
            pltpu.SemaphoreType.DMA,
            pltpu.SemaphoreType.DMA,
        ],
    )
    def agg_kernel(x_hbm, src_hbm, dst_hbm, zeros_hbm, out_hbm,
                   src_v, dst_v, rows0_v, rows1_v, acc_sh, sem0, sem1):
        c = lax.axis_index("c")
        s = lax.axis_index("s")
        wid = c * NS + s
        row0 = s * rows_per_tile

        pltpu.sync_copy(zeros_hbm, acc_sh.at[pl.ds(row0, rows_per_tile)])
        pltpu.sync_copy(src_hbm.at[wid], src_v)
        pltpu.sync_copy(dst_hbm.at[wid], dst_v)

        plsc.subcore_barrier()

        def gather(j, buf, sem):
            return pltpu.async_copy(
                x_hbm.at[src_v.at[pl.ds(j * batch, batch)]], buf, sem)

        def gather_wait(j, buf, sem):
            pltpu.make_async_copy(
                x_hbm.at[src_v.at[pl.ds(j * batch, batch)]], buf, sem).wait()

        gather(0, rows0_v, sem0)

        @pl.loop(0, n_iter - 1, step=2)
        def _edges(j):
            gather(j + 1, rows1_v, sem1)
            gather_wait(j, rows0_v, sem0)
            pltpu.sync_copy(rows0_v, acc_sh.at[dst_v.at[j]], add=True)
            gather(j + 2, rows0_v, sem0)
            gather_wait(j + 1, rows1_v, sem1)
            pltpu.sync_copy(rows1_v, acc_sh.at[dst_v.at[j + 1]], add=True)

        gather_wait(n_iter - 1, rows0_v, sem0)
        pltpu.sync_copy(rows0_v, acc_sh.at[dst_v.at[n_iter - 1]], add=True)

        plsc.subcore_barrier()
        pltpu.sync_copy(acc_sh.at[pl.ds(row0, rows_per_tile)],
                        out_hbm.at[c, pl.ds(row0, rows_per_tile)])

    return agg_kernel(x, src2, dst3, zeros_blk)


def _mlp_body(x_ref, a0_ref, a1_ref, wa_ref, ba_ref, wb_ref, bb_ref, o_ref):
    h = x_ref[...] + a0_ref[...] + a1_ref[...]
    h = jnp.maximum(
        jnp.dot(h, wa_ref[...], preferred_element_type=jnp.float32)
        + ba_ref[...], 0.0)
    h = jnp.maximum(
        jnp.dot(h, wb_ref[...], preferred_element_type=jnp.float32)
        + bb_ref[...], 0.0)
    o_ref[...] = h


def _mlp2_body(x_ref, a0_ref, a1_ref, wa_ref, ba_ref, wb_ref, bb_ref,
               wfc_ref, bfc_ref, o_ref):
    h = x_ref[...] + a0_ref[...] + a1_ref[...]
    h = jnp.maximum(
        jnp.dot(h, wa_ref[...], preferred_element_type=jnp.float32)
        + ba_ref[...], 0.0)
    h = jnp.maximum(
        jnp.dot(h, wb_ref[...], preferred_element_type=jnp.float32)
        + bb_ref[...], 0.0)
    logits = (jnp.dot(h, wfc_ref[...], preferred_element_type=jnp.float32)
              + bfc_ref[...])
    m = jnp.max(logits, axis=1, keepdims=True)
    lse = jnp.log(jnp.sum(jnp.exp(logits - m), axis=1, keepdims=True)) + m
    o_ref[...] = logits - lse


def _specs(bn, d, n_mats):
    row = pl.BlockSpec((bn, d), lambda i: (i, 0))
    mat = pl.BlockSpec((d, d), lambda i: (0, 0))
    vec = pl.BlockSpec((1, d), lambda i: (0, 0))
    return [row, row, row] + [mat, vec] * n_mats


def _mlp(x, a0, a1, wa, ba, wb, bb, bn):
    n, d = x.shape
    return pl.pallas_call(
        _mlp_body,
        grid=(n // bn,),
        in_specs=_specs(bn, d, 2),
        out_specs=pl.BlockSpec((bn, d), lambda i: (i, 0)),
        out_shape=jax.ShapeDtypeStruct((n, d), jnp.float32),
    )(x, a0, a1, wa, ba.reshape(1, d), wb, bb.reshape(1, d))


def _mlp2_fc_logsoftmax(x, a0, a1, wa, ba, wb, bb, wfc_p, bfc_p, bn):
    n, d = x.shape
    return pl.pallas_call(
        _mlp2_body,
        grid=(n // bn,),
        in_specs=_specs(bn, d, 3),
        out_specs=pl.BlockSpec((bn, d), lambda i: (i, 0)),
        out_shape=jax.ShapeDtypeStruct((n, d), jnp.float32),
    )(x, a0, a1, wa, ba.reshape(1, d), wb, bb.reshape(1, d),
      wfc_p, bfc_p.reshape(1, d))


def kernel(x, edge_index, W1a, b1a, W1b, b1b, W2a, b2a, W2b, b2b, Wfc, bfc):
    n, d = x.shape
    e = edge_index.shape[1]
    c = Wfc.shape[1]

    nt = NC * NS
    e_per_w = e // nt
    batch = 80
    n_iter = e_per_w // batch
    src2 = edge_index[0].reshape(nt, e_per_w)
    dst3 = edge_index[1].reshape(nt, n_iter, batch)
    n_pad = -(-n // (NS * 8)) * (NS * 8)
    zeros_blk = jnp.zeros((n_pad // NS, d), jnp.float32)

    wfc_p = jnp.zeros((d, d), jnp.float32).at[:, :c].set(Wfc)
    bfc_p = jnp.full((d,), -1e30, jnp.float32).at[:c].set(bfc)

    bn = 1000
    agg1 = _sc_aggregate(x, src2, dst3, zeros_blk)
    h1 = _mlp(x, agg1[0], agg1[1], W1a, b1a, W1b, b1b, bn)
    agg2 = _sc_aggregate(h1, src2, dst3, zeros_blk)
    out_p = _mlp2_fc_logsoftmax(h1, agg2[0], agg2[1], W2a, b2a, W2b, b2b,
                                wfc_p, bfc_p, bn)
    return out_p[:, :c]
